# baseline (speedup 1.0000x reference)
import functools

import jax
import jax.numpy as jnp
from jax import lax
from jax.experimental import pallas as pl
from jax.experimental.pallas import tpu as pltpu
from jax.experimental.pallas import tpu_sc as plsc

NC = 2
NS = 16
L = 16
NW = NC * NS


def _sc_partials_body(n_per_w, c_hbm, t_hbm, co_hbm, w_hbm, V_hbm, U_hbm,
                      vb_hbm, ub_hbm, out_hbm,
                      cidx, tidx, vrows, urows, vbv, ubv, cov, wv, pvec,
                      sem, semi, semg0, semg1):
    semg = [semg0, semg1]
    wid = lax.axis_index("s") * NC + lax.axis_index("c")
    base = wid * n_per_w

    cps = [
        pltpu.async_copy(co_hbm.at[pl.ds(base, n_per_w)], cov, sem),
        pltpu.async_copy(w_hbm.at[pl.ds(base, n_per_w)], wv, sem),
    ]
    ci = pltpu.async_copy(c_hbm.at[pl.ds(base, n_per_w)], cidx, semi)
    ti = pltpu.async_copy(t_hbm.at[pl.ds(base, n_per_w)], tidx, semi)
    ci.wait()
    ti.wait()
    ngrp = n_per_w // L
    grp = []
    for g in range(ngrp):
        sl = pl.ds(g * L, L)
        grp.append((
            pltpu.async_copy(V_hbm.at[cidx.at[sl]], vrows.at[sl], semg[g]),
            pltpu.async_copy(U_hbm.at[tidx.at[sl]], urows.at[sl], semg[g]),
        ))
    cps += [
        pltpu.async_copy(vb_hbm.at[cidx], vbv, sem),
        pltpu.async_copy(ub_hbm.at[tidx], ubv, sem),
    ]
    for cp in cps:
        cp.wait()

    io = lax.iota(jnp.int32, L)
    vs1 = jnp.zeros((L,), jnp.float32)
    vs2 = jnp.zeros((L,), jnp.float32)
    vs3 = jnp.zeros((L,), jnp.float32)
    vt1 = jnp.zeros((L,), jnp.float32)
    vt2 = jnp.zeros((L,), jnp.float32)
    for g in range(ngrp):
        for cp in grp[g]:
            cp.wait()
        def _row(jj, dots):
            j = g * L + jj

            def _chunk(k, acc):
                return acc + vrows[j, pl.ds(k * L, L)] * urows[j, pl.ds(k * L, L)]

            acc = lax.fori_loop(1, 128 // L, _chunk,
                                vrows[j, pl.ds(0, L)] * urows[j, pl.ds(0, L)])
            return jnp.where(io == jj, jnp.sum(acc), dots)
        dots = lax.fori_loop(0, L, _row, jnp.zeros((L,), jnp.float32))
        sl = pl.ds(g * L, L)
        a = dots - cov[sl]
        wg = wv[sl]
        b = vbv[sl] + ubv[sl]
        vs1 = vs1 + wg * a * a
        vs2 = vs2 + wg * a
        vs3 = vs3 + wg
        vt1 = vt1 + b
        vt2 = vt2 + b * b

    pvec[0, :] = vs1
    pvec[1, :] = vs2
    pvec[2, :] = vs3
    pvec[3, :] = vt1
    pvec[4, :] = vt2
    pltpu.sync_copy(pvec, out_hbm.at[wid])


def _combine_body(nb, p_ref, o_ref):
    p = p_ref[:, :, :]
    s1 = jnp.sum(p[:, 0, :])
    s2 = jnp.sum(p[:, 1, :])
    s3 = jnp.sum(p[:, 2, :])
    t1 = jnp.sum(p[:, 3, :])
    t2 = jnp.sum(p[:, 4, :])
    o_ref[0, 0] = nb * s1 + 2.0 * s2 * t1 + s3 * t2


def kernel(center_words, target_words, coocs, weights, V, U, Vb, Ub):
    b_total = center_words.shape[0]
    n_per_w = b_total // NW
    c = center_words.reshape(b_total).astype(jnp.int32)
    t = target_words.reshape(b_total).astype(jnp.int32)
    co = coocs.reshape(b_total)
    w = weights.reshape(b_total)
    vb = Vb.reshape(Vb.shape[0])
    ub = Ub.reshape(Ub.shape[0])

    mesh = plsc.VectorSubcoreMesh(core_axis_name="c", subcore_axis_name="s")
    sc_call = pl.kernel(
        functools.partial(_sc_partials_body, n_per_w),
        out_type=jax.ShapeDtypeStruct((NW, 5, L), jnp.float32),
        mesh=mesh,
        compiler_params=pltpu.CompilerParams(needs_layout_passes=False),
        scratch_types=[
            pltpu.VMEM((n_per_w,), jnp.int32),
            pltpu.VMEM((n_per_w,), jnp.int32),
            pltpu.VMEM((n_per_w, 128), jnp.float32),
            pltpu.VMEM((n_per_w, 128), jnp.float32),
            pltpu.VMEM((n_per_w,), jnp.float32),
            pltpu.VMEM((n_per_w,), jnp.float32),
            pltpu.VMEM((n_per_w,), jnp.float32),
            pltpu.VMEM((n_per_w,), jnp.float32),
            pltpu.VMEM((5, L), jnp.float32),
            pltpu.SemaphoreType.DMA,
            pltpu.SemaphoreType.DMA,
            pltpu.SemaphoreType.DMA,
            pltpu.SemaphoreType.DMA,
        ],
    )
    partials = sc_call(c, t, co, w, V, U, vb, ub)

    loss = pl.pallas_call(
        functools.partial(_combine_body, float(b_total)),
        out_shape=jax.ShapeDtypeStruct((1, 1), jnp.float32),
        out_specs=pl.BlockSpec(memory_space=pltpu.SMEM),
    )(partials)
    return loss[0, 0]


# speedup vs baseline: 1.0238x; 1.0238x over previous
import functools

import jax
import jax.numpy as jnp
from jax import lax
from jax.experimental import pallas as pl
from jax.experimental.pallas import tpu as pltpu
from jax.experimental.pallas import tpu_sc as plsc

NC = 1
NS = 16
L = 16
NW = NC * NS


def _sc_partials_body(n_per_w, c_hbm, t_hbm, co_hbm, w_hbm, V_hbm, U_hbm,
                      vb_hbm, ub_hbm, out_hbm,
                      cidx, tidx, vrows, urows, vbv, ubv, cov, wv, pvec,
                      sem, semi, semg0, semg1, semg2, semg3):
    semg = [semg0, semg1, semg2, semg3]
    wid = lax.axis_index("s") * NC + lax.axis_index("c")
    base = wid * n_per_w

    cps = [
        pltpu.async_copy(co_hbm.at[pl.ds(base, n_per_w)], cov, sem),
        pltpu.async_copy(w_hbm.at[pl.ds(base, n_per_w)], wv, sem),
    ]
    ci = pltpu.async_copy(c_hbm.at[pl.ds(base, n_per_w)], cidx, semi)
    ti = pltpu.async_copy(t_hbm.at[pl.ds(base, n_per_w)], tidx, semi)
    ci.wait()
    ti.wait()
    ngrp = n_per_w // L
    grp = []
    for g in range(ngrp):
        sl = pl.ds(g * L, L)
        grp.append((
            pltpu.async_copy(V_hbm.at[cidx.at[sl]], vrows.at[sl], semg[g]),
            pltpu.async_copy(U_hbm.at[tidx.at[sl]], urows.at[sl], semg[g]),
        ))
    cps += [
        pltpu.async_copy(vb_hbm.at[cidx], vbv, sem),
        pltpu.async_copy(ub_hbm.at[tidx], ubv, sem),
    ]
    for cp in cps:
        cp.wait()

    io = lax.iota(jnp.int32, L)
    vs1 = jnp.zeros((L,), jnp.float32)
    vs2 = jnp.zeros((L,), jnp.float32)
    vs3 = jnp.zeros((L,), jnp.float32)
    vt1 = jnp.zeros((L,), jnp.float32)
    vt2 = jnp.zeros((L,), jnp.float32)
    for g in range(ngrp):
        for cp in grp[g]:
            cp.wait()
        def _row(jj, dots):
            j = g * L + jj

            def _chunk(k, acc):
                return acc + vrows[j, pl.ds(k * L, L)] * urows[j, pl.ds(k * L, L)]

            acc = lax.fori_loop(1, 128 // L, _chunk,
                                vrows[j, pl.ds(0, L)] * urows[j, pl.ds(0, L)])
            return jnp.where(io == jj, jnp.sum(acc), dots)
        dots = lax.fori_loop(0, L, _row, jnp.zeros((L,), jnp.float32))
        sl = pl.ds(g * L, L)
        a = dots - cov[sl]
        wg = wv[sl]
        b = vbv[sl] + ubv[sl]
        vs1 = vs1 + wg * a * a
        vs2 = vs2 + wg * a
        vs3 = vs3 + wg
        vt1 = vt1 + b
        vt2 = vt2 + b * b

    pvec[0, :] = vs1
    pvec[1, :] = vs2
    pvec[2, :] = vs3
    pvec[3, :] = vt1
    pvec[4, :] = vt2
    pltpu.sync_copy(pvec, out_hbm.at[wid])


def _combine_body(nb, p_ref, o_ref):
    p = p_ref[:, :, :]
    s1 = jnp.sum(p[:, 0, :])
    s2 = jnp.sum(p[:, 1, :])
    s3 = jnp.sum(p[:, 2, :])
    t1 = jnp.sum(p[:, 3, :])
    t2 = jnp.sum(p[:, 4, :])
    o_ref[0, 0] = nb * s1 + 2.0 * s2 * t1 + s3 * t2


def kernel(center_words, target_words, coocs, weights, V, U, Vb, Ub):
    b_total = center_words.shape[0]
    n_per_w = b_total // NW
    c = center_words.reshape(b_total).astype(jnp.int32)
    t = target_words.reshape(b_total).astype(jnp.int32)
    co = coocs.reshape(b_total)
    w = weights.reshape(b_total)
    vb = Vb.reshape(Vb.shape[0])
    ub = Ub.reshape(Ub.shape[0])

    mesh = plsc.VectorSubcoreMesh(core_axis_name="c", subcore_axis_name="s",
                                  num_cores=NC)
    sc_call = pl.kernel(
        functools.partial(_sc_partials_body, n_per_w),
        out_type=jax.ShapeDtypeStruct((NW, 5, L), jnp.float32),
        mesh=mesh,
        compiler_params=pltpu.CompilerParams(needs_layout_passes=False),
        scratch_types=[
            pltpu.VMEM((n_per_w,), jnp.int32),
            pltpu.VMEM((n_per_w,), jnp.int32),
            pltpu.VMEM((n_per_w, 128), jnp.float32),
            pltpu.VMEM((n_per_w, 128), jnp.float32),
            pltpu.VMEM((n_per_w,), jnp.float32),
            pltpu.VMEM((n_per_w,), jnp.float32),
            pltpu.VMEM((n_per_w,), jnp.float32),
            pltpu.VMEM((n_per_w,), jnp.float32),
            pltpu.VMEM((5, L), jnp.float32),
            pltpu.SemaphoreType.DMA,
            pltpu.SemaphoreType.DMA,
            pltpu.SemaphoreType.DMA,
            pltpu.SemaphoreType.DMA,
            pltpu.SemaphoreType.DMA,
            pltpu.SemaphoreType.DMA,
        ],
    )
    partials = sc_call(c, t, co, w, V, U, vb, ub)

    loss = pl.pallas_call(
        functools.partial(_combine_body, float(b_total)),
        out_shape=jax.ShapeDtypeStruct((1, 1), jnp.float32),
        out_specs=pl.BlockSpec(memory_space=pltpu.SMEM),
    )(partials)
    return loss[0, 0]


# speedup vs baseline: 1.0657x; 1.0410x over previous
import functools

import jax
import jax.numpy as jnp
from jax import lax
from jax.experimental import pallas as pl
from jax.experimental.pallas import tpu as pltpu
from jax.experimental.pallas import tpu_sc as plsc

NC = 1
NS = 16
L = 16
NW = NC * NS


def _sc_partials_body(n_per_w, nb, c_hbm, t_hbm, co_hbm, w_hbm, V_hbm, U_hbm,
                      vb_hbm, ub_hbm, out_hbm,
                      cidx, tidx, vrows, urows, vbv, ubv, cov, wv, pvec,
                      shp, allp, lvec,
                      sem, semi, semg0, semg1, semg2, semg3):
    semg = [semg0, semg1, semg2, semg3]
    wid = lax.axis_index("s") * NC + lax.axis_index("c")
    base = wid * n_per_w

    cps = [
        pltpu.async_copy(co_hbm.at[pl.ds(base, n_per_w)], cov, sem),
        pltpu.async_copy(w_hbm.at[pl.ds(base, n_per_w)], wv, sem),
    ]
    ci = pltpu.async_copy(c_hbm.at[pl.ds(base, n_per_w)], cidx, semi)
    ti = pltpu.async_copy(t_hbm.at[pl.ds(base, n_per_w)], tidx, semi)
    ci.wait()
    ti.wait()
    ngrp = n_per_w // L
    grp = []
    for g in range(ngrp):
        sl = pl.ds(g * L, L)
        grp.append((
            pltpu.async_copy(V_hbm.at[cidx.at[sl]], vrows.at[sl], semg[g]),
            pltpu.async_copy(U_hbm.at[tidx.at[sl]], urows.at[sl], semg[g]),
        ))
    cps += [
        pltpu.async_copy(vb_hbm.at[cidx], vbv, sem),
        pltpu.async_copy(ub_hbm.at[tidx], ubv, sem),
    ]
    for cp in cps:
        cp.wait()

    io = lax.iota(jnp.int32, L)
    vs1 = jnp.zeros((L,), jnp.float32)
    vs2 = jnp.zeros((L,), jnp.float32)
    vs3 = jnp.zeros((L,), jnp.float32)
    vt1 = jnp.zeros((L,), jnp.float32)
    vt2 = jnp.zeros((L,), jnp.float32)
    for g in range(ngrp):
        for cp in grp[g]:
            cp.wait()
        def _row(jj, dots):
            j = g * L + jj

            def _chunk(k, acc):
                return acc + vrows[j, pl.ds(k * L, L)] * urows[j, pl.ds(k * L, L)]

            acc = lax.fori_loop(1, 128 // L, _chunk,
                                vrows[j, pl.ds(0, L)] * urows[j, pl.ds(0, L)])
            return jnp.where(io == jj, jnp.sum(acc), dots)
        dots = lax.fori_loop(0, L, _row, jnp.zeros((L,), jnp.float32))
        sl = pl.ds(g * L, L)
        a = dots - cov[sl]
        wg = wv[sl]
        b = vbv[sl] + ubv[sl]
        vs1 = vs1 + wg * a * a
        vs2 = vs2 + wg * a
        vs3 = vs3 + wg
        vt1 = vt1 + b
        vt2 = vt2 + b * b

    pvec[0, :] = vs1
    pvec[1, :] = vs2
    pvec[2, :] = vs3
    pvec[3, :] = vt1
    pvec[4, :] = vt2
    pltpu.sync_copy(pvec, shp.at[pl.ds(wid * 5, 5)])
    plsc.subcore_barrier()

    @pl.when(wid == 0)
    def _fold():
        pltpu.sync_copy(shp, allp)
        av = [jnp.zeros((L,), jnp.float32) for _ in range(5)]
        for t in range(NS):
            for q in range(5):
                av[q] = av[q] + allp[t * 5 + q, :]
        ss = [jnp.full((L,), jnp.sum(av[q]), jnp.float32) for q in range(5)]
        lossv = nb * ss[0] + 2.0 * ss[1] * ss[3] + ss[2] * ss[4]
        lvec[...] = lossv
        pltpu.sync_copy(lvec, out_hbm)


def kernel(center_words, target_words, coocs, weights, V, U, Vb, Ub):
    b_total = center_words.shape[0]
    n_per_w = b_total // NW
    c = center_words.reshape(b_total).astype(jnp.int32)
    t = target_words.reshape(b_total).astype(jnp.int32)
    co = coocs.reshape(b_total)
    w = weights.reshape(b_total)
    vb = Vb.reshape(Vb.shape[0])
    ub = Ub.reshape(Ub.shape[0])

    mesh = plsc.VectorSubcoreMesh(core_axis_name="c", subcore_axis_name="s",
                                  num_cores=NC)
    sc_call = pl.kernel(
        functools.partial(_sc_partials_body, n_per_w, float(b_total)),
        out_type=jax.ShapeDtypeStruct((L,), jnp.float32),
        mesh=mesh,
        compiler_params=pltpu.CompilerParams(needs_layout_passes=False),
        scratch_types=[
            pltpu.VMEM((n_per_w,), jnp.int32),
            pltpu.VMEM((n_per_w,), jnp.int32),
            pltpu.VMEM((n_per_w, 128), jnp.float32),
            pltpu.VMEM((n_per_w, 128), jnp.float32),
            pltpu.VMEM((n_per_w,), jnp.float32),
            pltpu.VMEM((n_per_w,), jnp.float32),
            pltpu.VMEM((n_per_w,), jnp.float32),
            pltpu.VMEM((n_per_w,), jnp.float32),
            pltpu.VMEM((5, L), jnp.float32),
            pltpu.VMEM_SHARED((NW * 5, L), jnp.float32),
            pltpu.VMEM((NW * 5, L), jnp.float32),
            pltpu.VMEM((L,), jnp.float32),
            pltpu.SemaphoreType.DMA,
            pltpu.SemaphoreType.DMA,
            pltpu.SemaphoreType.DMA,
            pltpu.SemaphoreType.DMA,
            pltpu.SemaphoreType.DMA,
            pltpu.SemaphoreType.DMA,
        ],
    )
    out = sc_call(c, t, co, w, V, U, vb, ub)
    return out[0]
